# baseline (speedup 1.0000x reference)
import jax
import jax.numpy as jnp
from jax.experimental import pallas as pl
from jax.experimental.pallas import tpu as pltpu
from jax.experimental.pallas import tpu_sc as plsc

_N_VOCAB = 100000
_EMBED_DIM = 32
_BATCH = 16384
_HIST = 200
_N_IDX = _BATCH * _HIST

_WINDOW = 1024


def _gather_impl(emb, idx2d):
    mesh = plsc.VectorSubcoreMesh(core_axis_name="core",
                                  subcore_axis_name="subcore")

    @pl.kernel(
        out_type=jax.ShapeDtypeStruct((_N_IDX, _EMBED_DIM), jnp.float32),
        mesh=mesh,
        compiler_params=pltpu.CompilerParams(use_tc_tiling_on_sc=False),
    )
    def k(emb_hbm, idx_hbm, out_hbm):
        def body(i_vmem, o_vmem):
            pltpu.sync_copy(emb_hbm.at[i_vmem.at[0]], o_vmem)

        pltpu.emit_pipeline(
            body,
            grid=(_N_IDX // _WINDOW,),
            in_specs=[pl.BlockSpec((1, _WINDOW), index_map=lambda i: (0, i))],
            out_specs=[pl.BlockSpec((_WINDOW, _EMBED_DIM),
                                    index_map=lambda i: (i, 0))],
            core_axis_name=("core", "subcore"),
            dimension_semantics=(pltpu.PARALLEL,),
        )(idx_hbm, out_hbm)

    return k(emb, idx2d)


def kernel(a_in, emb):
    idx = a_in.astype(jnp.int32).reshape(1, _N_IDX)
    out = _gather_impl(emb, idx)
    return out.reshape(_BATCH, _HIST, _EMBED_DIM)


# speedup vs baseline: 1.0055x; 1.0055x over previous
import jax
import jax.numpy as jnp
from jax.experimental import pallas as pl
from jax.experimental.pallas import tpu as pltpu
from jax.experimental.pallas import tpu_sc as plsc

_N_VOCAB = 100000
_EMBED_DIM = 32
_BATCH = 16384
_HIST = 200

_ROWS = 8
_WINDOW = _ROWS * _HIST


def _gather_impl(emb, idx):
    mesh = plsc.VectorSubcoreMesh(core_axis_name="core",
                                  subcore_axis_name="subcore")

    @pl.kernel(
        out_type=jax.ShapeDtypeStruct((_BATCH, _HIST, _EMBED_DIM),
                                      jnp.float32),
        mesh=mesh,
        scratch_types=[pltpu.SemaphoreType.DMA],
        compiler_params=pltpu.CompilerParams(use_tc_tiling_on_sc=False),
    )
    def k(emb_hbm, idx_hbm, out_hbm, sem):
        def body(i_vmem, o_vmem):
            copies = [
                pltpu.async_copy(emb_hbm.at[i_vmem.at[j]], o_vmem.at[j], sem)
                for j in range(_ROWS)
            ]
            for c in copies:
                c.wait()

        pltpu.emit_pipeline(
            body,
            grid=(_BATCH // _ROWS,),
            in_specs=[pl.BlockSpec((_ROWS, _HIST), index_map=lambda i: (i, 0))],
            out_specs=[pl.BlockSpec((_ROWS, _HIST, _EMBED_DIM),
                                    index_map=lambda i: (i, 0, 0))],
            core_axis_name=("core", "subcore"),
            dimension_semantics=(pltpu.PARALLEL,),
        )(idx_hbm, out_hbm)

    return k(emb, idx)


def kernel(a_in, emb):
    return _gather_impl(emb, a_in.astype(jnp.int32))


# speedup vs baseline: 1.0067x; 1.0012x over previous
import jax
import jax.numpy as jnp
from jax import lax
from jax.experimental import pallas as pl
from jax.experimental.pallas import tpu as pltpu
from jax.experimental.pallas import tpu_sc as plsc

_N_VOCAB = 100000
_EMBED_DIM = 32
_BATCH = 16384
_HIST = 200

_LANES = 16
_BT = _BATCH // 128
_DT = _EMBED_DIM // 8


def _gather_impl(emb, idx_t):
    mesh = plsc.VectorSubcoreMesh(core_axis_name="core",
                                  subcore_axis_name="subcore")

    @pl.kernel(
        out_type=jax.ShapeDtypeStruct((_HIST, _DT, _BT, 8, 128),
                                      jnp.float32),
        mesh=mesh,
        scratch_types=[pltpu.VMEM((128, _EMBED_DIM), jnp.float32)],
        compiler_params=pltpu.CompilerParams(use_tc_tiling_on_sc=False,
                                             needs_layout_passes=False),
    )
    def k(emb_hbm, idx_hbm, out_hbm, buf):
        def body(i_vmem, o_vmem):
            pltpu.sync_copy(emb_hbm.at[i_vmem.at[0]], buf)
            iota = lax.iota(jnp.int32, _LANES)

            @plsc.parallel_loop(0, 256, unroll=8)
            def _transpose(i):
                g = lax.rem(i, 8)
                d = lax.div(i, 8)
                row = g * _LANES + iota
                col = jnp.full((_LANES,), 1, jnp.int32) * d
                vals = plsc.load_gather(buf, [row, col])
                o_vmem[0, lax.div(d, 8), 0, lax.rem(d, 8),
                       pl.ds(g * _LANES, _LANES)] = vals

        pltpu.emit_pipeline(
            body,
            grid=(_HIST * _BT,),
            in_specs=[pl.BlockSpec((1, 128),
                                   index_map=lambda s: (s // _BT, s % _BT))],
            out_specs=[pl.BlockSpec((1, _DT, 1, 8, 128),
                                    index_map=lambda s: (s // _BT, 0,
                                                         s % _BT, 0, 0))],
            core_axis_name=("core", "subcore"),
            dimension_semantics=(pltpu.PARALLEL,),
        )(idx_hbm, out_hbm)

    return k(emb, idx_t)


def kernel(a_in, emb):
    idx_t = a_in.T.astype(jnp.int32)
    x5 = _gather_impl(emb, idx_t)
    return x5.transpose(2, 4, 0, 1, 3).reshape(_BATCH, _HIST, _EMBED_DIM)


# speedup vs baseline: 1.3151x; 1.3064x over previous
import jax
import jax.numpy as jnp
from jax import lax
from jax.experimental import pallas as pl
from jax.experimental.pallas import tpu as pltpu
from jax.experimental.pallas import tpu_sc as plsc

_N_VOCAB = 100000
_EMBED_DIM = 32
_BATCH = 16384
_HIST = 200

_LANES = 16
_BT = _BATCH // 128
_DT = _EMBED_DIM // 8
_K = 4
_SBT = _BT // _K


def _gather_impl(emb, idx_t):
    mesh = plsc.VectorSubcoreMesh(core_axis_name="core",
                                  subcore_axis_name="subcore")

    @pl.kernel(
        out_type=jax.ShapeDtypeStruct((_HIST, _DT, _BT, 8, 128),
                                      jnp.float32),
        mesh=mesh,
        scratch_types=[pltpu.VMEM((_K, 128, _EMBED_DIM), jnp.float32),
                       pltpu.SemaphoreType.DMA],
        compiler_params=pltpu.CompilerParams(use_tc_tiling_on_sc=False,
                                             needs_layout_passes=False),
    )
    def k(emb_hbm, idx_hbm, out_hbm, buf, sem):
        def body(i_vmem, o_vmem):
            copies = [
                pltpu.async_copy(emb_hbm.at[i_vmem.at[0, pl.ds(j * 128, 128)]],
                                 buf.at[j], sem)
                for j in range(_K)
            ]
            iota = lax.iota(jnp.int32, _LANES)
            rows = [iota + g * _LANES for g in range(128 // _LANES)]
            ones = jnp.full((_LANES,), 1, jnp.int32)
            for j in range(_K):
                copies[j].wait()

                @plsc.parallel_loop(0, _EMBED_DIM, unroll=4)
                def _transpose(d):
                    dt = lax.div(d, 8)
                    sub = lax.rem(d, 8)
                    col = ones * d
                    for g in range(128 // _LANES):
                        vals = plsc.load_gather(buf.at[j], [rows[g], col])
                        o_vmem[0, dt, j, sub,
                               pl.ds(g * _LANES, _LANES)] = vals

        pltpu.emit_pipeline(
            body,
            grid=(_HIST * _SBT,),
            in_specs=[pl.BlockSpec((1, 128 * _K),
                                   index_map=lambda s: (s // _SBT, s % _SBT))],
            out_specs=[pl.BlockSpec((1, _DT, _K, 8, 128),
                                    index_map=lambda s: (s // _SBT, 0,
                                                         s % _SBT, 0, 0))],
            core_axis_name=("core", "subcore"),
            dimension_semantics=(pltpu.PARALLEL,),
        )(idx_hbm, out_hbm)

    return k(emb, idx_t)


def kernel(a_in, emb):
    idx_t = a_in.T.astype(jnp.int32)
    x5 = _gather_impl(emb, idx_t)
    return x5.transpose(2, 4, 0, 1, 3).reshape(_BATCH, _HIST, _EMBED_DIM)


# speedup vs baseline: 4.2284x; 3.2152x over previous
import jax
import jax.numpy as jnp
from jax import lax
from jax.experimental import pallas as pl
from jax.experimental.pallas import tpu as pltpu
from jax.experimental.pallas import tpu_sc as plsc

_N_VOCAB = 100000
_EMBED_DIM = 32
_BATCH = 16384
_HIST = 200

_LANES = 16
_BT = _BATCH // 128
_DT = _EMBED_DIM // 8
_K = 4
_SBT = _BT // _K


def _gather_impl(emb, idx_t):
    mesh = plsc.VectorSubcoreMesh(core_axis_name="core",
                                  subcore_axis_name="subcore")

    @pl.kernel(
        out_type=jax.ShapeDtypeStruct((_HIST, _DT, _BT, 8, 128),
                                      jnp.float32),
        mesh=mesh,
        scratch_types=[pltpu.VMEM((_K, 128, _EMBED_DIM), jnp.float32),
                       pltpu.SemaphoreType.DMA],
        compiler_params=pltpu.CompilerParams(use_tc_tiling_on_sc=False,
                                             needs_layout_passes=False),
    )
    def k(emb_hbm, idx_hbm, out_hbm, buf, sem):
        def body(i_vmem, o_vmem):
            copies = [
                pltpu.async_copy(emb_hbm.at[i_vmem.at[0, pl.ds(j * 128, 128)]],
                                 buf.at[j], sem)
                for j in range(_K)
            ]
            iota = lax.iota(jnp.int32, _LANES)
            lanes = [iota + g * _LANES for g in range(128 // _LANES)]
            zero = jnp.full((_LANES,), 0, jnp.int32)
            ones = jnp.full((_LANES,), 1, jnp.int32)
            for j in range(_K):
                copies[j].wait()
                jvec = ones * j

                @plsc.parallel_loop(0, _EMBED_DIM, unroll=4)
                def _transpose(d):
                    dd = lax.bitwise_and(d + iota, _EMBED_DIM - 1)
                    dtv = lax.shift_right_logical(dd, 3)
                    subv = lax.bitwise_and(dd, 7)
                    for g in range(128 // _LANES):
                        vals = plsc.load_gather(buf.at[j], [lanes[g], dd])
                        plsc.store_scatter(
                            o_vmem, [zero, dtv, jvec, subv, lanes[g]], vals)

        pltpu.emit_pipeline(
            body,
            grid=(_HIST * _SBT,),
            in_specs=[pl.BlockSpec((1, 128 * _K),
                                   index_map=lambda s: (s // _SBT, s % _SBT))],
            out_specs=[pl.BlockSpec((1, _DT, _K, 8, 128),
                                    index_map=lambda s: (s // _SBT, 0,
                                                         s % _SBT, 0, 0))],
            core_axis_name=("core", "subcore"),
            dimension_semantics=(pltpu.PARALLEL,),
        )(idx_hbm, out_hbm)

    return k(emb, idx_t)


def kernel(a_in, emb):
    idx_t = a_in.T.astype(jnp.int32)
    x5 = _gather_impl(emb, idx_t)
    return x5.transpose(2, 4, 0, 1, 3).reshape(_BATCH, _HIST, _EMBED_DIM)
